# baseline (speedup 1.0000x reference)
import functools

import jax
import jax.numpy as jnp
from jax import lax
from jax.experimental import pallas as pl
from jax.experimental.pallas import tpu as pltpu
from jax.experimental.pallas import tpu_sc as plsc

N_USERS = 1000000
N_ITEMS = 1000000
EMBED_DIM = 32
BATCH = 16384

NC = 2
NS = 16
L = 16
NW = NC * NS
B_PER_W = BATCH // NW
N_CHUNKS = B_PER_W // L


def _gmf_body(users_hbm, items_hbm, utab_hbm, itab_hbm, w_hbm, b_hbm,
              out_hbm, idx_u, idx_i, rows_u, rows_i, wv, bv, out_v,
              sem_u, sem_i):
    wid = lax.axis_index("s") * NC + lax.axis_index("c")
    base = wid * B_PER_W

    pltpu.sync_copy(users_hbm.at[pl.ds(base, B_PER_W)], idx_u)
    pltpu.sync_copy(items_hbm.at[pl.ds(base, B_PER_W)], idx_i)
    pltpu.sync_copy(w_hbm, wv)
    pltpu.sync_copy(b_hbm, bv.at[pl.ds(0, 1)])

    cp_u = pltpu.async_copy(utab_hbm.at[idx_u], rows_u, sem_u)
    cp_i = pltpu.async_copy(itab_hbm.at[idx_i], rows_i, sem_i)
    cp_u.wait()
    cp_i.wait()

    iota16 = lax.iota(jnp.int32, L)
    b_s = bv[pl.ds(0, L)][0]
    w_lo = wv[pl.ds(0, L)]
    w_hi = wv[pl.ds(L, L)]
    w_s = [w_lo[d] for d in range(L)] + [w_hi[d] for d in range(L)]

    def chunk_body(c, _):
        row_idx = iota16 + c * L
        acc = jnp.full((L,), b_s, dtype=jnp.float32)
        for d in range(EMBED_DIM):
            col = jnp.full((L,), d, dtype=jnp.int32)
            u_g = plsc.load_gather(rows_u, [row_idx, col])
            i_g = plsc.load_gather(rows_i, [row_idx, col])
            acc = acc + (u_g * i_g) * w_s[d]
        out_v[pl.ds(c * L, L)] = acc
        return ()

    lax.fori_loop(0, N_CHUNKS, chunk_body, (), unroll=False)

    pltpu.sync_copy(out_v, out_hbm.at[pl.ds(base, B_PER_W)])


@jax.jit
def _gmf(users, items, user_table, item_table, w, b):
    mesh = plsc.VectorSubcoreMesh(core_axis_name="c", subcore_axis_name="s")
    run = functools.partial(
        pl.kernel,
        mesh=mesh,
        out_type=jax.ShapeDtypeStruct((BATCH,), jnp.float32),
        compiler_params=pltpu.CompilerParams(
            needs_layout_passes=False, use_tc_tiling_on_sc=False),
        scratch_types=[
            pltpu.VMEM((B_PER_W,), jnp.int32),
            pltpu.VMEM((B_PER_W,), jnp.int32),
            pltpu.VMEM((B_PER_W, EMBED_DIM), jnp.float32),
            pltpu.VMEM((B_PER_W, EMBED_DIM), jnp.float32),
            pltpu.VMEM((EMBED_DIM,), jnp.float32),
            pltpu.VMEM((L,), jnp.float32),
            pltpu.VMEM((B_PER_W,), jnp.float32),
            pltpu.SemaphoreType.DMA,
            pltpu.SemaphoreType.DMA,
        ],
    )(_gmf_body)
    return run(users, items, user_table, item_table, w, b)


def kernel(users, items, user_table, item_table, W, b):
    w_flat = W.reshape(EMBED_DIM)
    return _gmf(users, items, user_table, item_table, w_flat, b)
